# baseline (speedup 1.0000x reference)
import jax
import jax.numpy as jnp
from jax import lax
from jax.experimental import pallas as pl
from jax.experimental.pallas import tpu as pltpu
from jax.experimental.pallas import tpu_sc as plsc

N = 10000
D = 128
GAMMA = 3.0
T = 1000
K = 5

BM = 400

P_PAIRS = T + K * T + K * T + T + T
NW = 32
B_PER_W = 416
P_PAD = NW * B_PER_W
CHUNK = 104
NCHUNK = B_PER_W // CHUNK

CHUNKS = [(0, 2048), (2048, 2048), (4096, 2048), (6144, 2048), (8192, 1808)]


def _mm_body(s_ref, w_ref, o_ref, hidden_ref):
    p = pl.program_id(0)
    mi = pl.program_id(1)

    @pl.when(p == 0)
    def _():
        h = jnp.zeros((BM, D), jnp.float32)
        for c0, cw in CHUNKS:
            s_c = s_ref[:, c0:c0 + cw].astype(jnp.bfloat16)
            w_c = w_ref[c0:c0 + cw, :].astype(jnp.bfloat16)
            h = h + jnp.dot(s_c, w_c, preferred_element_type=jnp.float32)
        hidden_ref[pl.ds(mi * BM, BM), :] = jnp.maximum(
            h, 0.0).astype(jnp.bfloat16)

    @pl.when(p == 1)
    def _():
        o = jnp.zeros((BM, D), jnp.float32)
        for c0, cw in CHUNKS:
            s_c = s_ref[:, c0:c0 + cw].astype(jnp.bfloat16)
            h_c = hidden_ref[c0:c0 + cw, :]
            o = o + jnp.dot(s_c, h_c, preferred_element_type=jnp.float32)
        o_ref[...] = o


def _gcn_out(support, W0):
    return pl.pallas_call(
        _mm_body,
        grid=(2, N // BM),
        in_specs=[
            pl.BlockSpec((BM, N), lambda p, mi: (mi, 0)),
            pl.BlockSpec((N, D), lambda p, mi: (0, 0)),
        ],
        out_specs=pl.BlockSpec((BM, D), lambda p, mi: (mi, 0)),
        out_shape=jax.ShapeDtypeStruct((N, D), jnp.float32),
        scratch_shapes=[
            pltpu.VMEM((N, D), jnp.bfloat16),
        ],
    )(support, W0)


def _sc_pair_dists(out_hbm, left, right):
    mesh = plsc.VectorSubcoreMesh(core_axis_name="c", subcore_axis_name="s")

    @pl.kernel(
        out_type=jax.ShapeDtypeStruct((P_PAD, 16), jnp.float32),
        mesh=mesh,
        scratch_types=[
            pltpu.VMEM((B_PER_W,), jnp.int32),
            pltpu.VMEM((B_PER_W,), jnp.int32),
            pltpu.VMEM((CHUNK, D), jnp.float32),
            pltpu.VMEM((CHUNK, D), jnp.float32),
            pltpu.VMEM((CHUNK, D), jnp.float32),
            pltpu.VMEM((CHUNK, D), jnp.float32),
            pltpu.VMEM((B_PER_W, 16), jnp.float32),
            pltpu.SemaphoreType.DMA,
            pltpu.SemaphoreType.DMA,
        ],
    )
    def k(x_hbm, li_hbm, ri_hbm, d_hbm, li_v, ri_v, a0_v, b0_v, a1_v, b1_v,
          d_v, sem0, sem1):
        wid = lax.axis_index("s") * 2 + lax.axis_index("c")
        base = wid * B_PER_W
        pltpu.sync_copy(li_hbm.at[pl.ds(base, B_PER_W)], li_v)
        pltpu.sync_copy(ri_hbm.at[pl.ds(base, B_PER_W)], ri_v)

        bufs = [(a0_v, b0_v, sem0), (a1_v, b1_v, sem1)]

        def issue(c):
            a_v, b_v, sem = bufs[c % 2]
            ca = pltpu.async_copy(
                x_hbm.at[li_v.at[pl.ds(c * CHUNK, CHUNK)]], a_v, sem)
            cb = pltpu.async_copy(
                x_hbm.at[ri_v.at[pl.ds(c * CHUNK, CHUNK)]], b_v, sem)
            return ca, cb

        def compute(c):
            a_v, b_v, _ = bufs[c % 2]

            @pl.loop(0, CHUNK)
            def _(r):
                acc = jnp.zeros((16,), jnp.float32)
                for j in range(D // 16):
                    av = a_v[r, pl.ds(j * 16, 16)]
                    bv = b_v[r, pl.ds(j * 16, 16)]
                    acc = acc + jnp.abs(av - bv)
                d_v[c * CHUNK + r, :] = acc

        pend = issue(0)
        for c in range(NCHUNK):
            pend[0].wait()
            pend[1].wait()
            if c + 1 < NCHUNK:
                pend = issue(c + 1)
            compute(c)

        pltpu.sync_copy(d_v, d_hbm.at[pl.ds(base, B_PER_W)])

    return k(out_hbm, left, right)


def _combine_body(d_ref, o_ref):
    d = jnp.sum(d_ref[...], axis=1, keepdims=True)
    dA = d[0:T]
    dA2 = d[11 * T:12 * T]
    dB3 = d[12 * T:13 * T]
    acc = jnp.sum(jnp.maximum(dA2 + GAMMA - dB3, 0.0))
    for k in range(2 * K):
        dBk = d[(1 + k) * T:(2 + k) * T]
        acc = acc + jnp.sum(jnp.maximum(dA + GAMMA - dBk, 0.0))
    o_ref[0, 0] = acc / (2 * K * T + T)


def _combine(dists):
    return pl.pallas_call(
        _combine_body,
        out_shape=jax.ShapeDtypeStruct((1, 1), jnp.float32),
        out_specs=pl.BlockSpec(memory_space=pltpu.SMEM),
    )(dists)


def kernel(features, support, W0, ILL0, ILL1, neg_left, neg_right,
           neg2_left, neg2_right, feedback_neg_left, feedback_neg_right,
           feedback_pos_left, feedback_pos_right):
    out = _gcn_out(support, W0)
    nl = neg_left.reshape(T, K).T.reshape(K * T)
    nr = neg_right.reshape(T, K).T.reshape(K * T)
    n2l = neg2_left.reshape(T, K).T.reshape(K * T)
    n2r = neg2_right.reshape(T, K).T.reshape(K * T)
    pad = jnp.zeros((P_PAD - P_PAIRS,), jnp.int32)
    left = jnp.concatenate([
        ILL0, nl, n2l, feedback_pos_left, feedback_neg_left,
        pad]).astype(jnp.int32)
    right = jnp.concatenate([
        ILL1, nr, n2r, feedback_pos_right, feedback_neg_right,
        pad]).astype(jnp.int32)
    dists = _sc_pair_dists(out, left, right)
    return _combine(dists)[0, 0]


# speedup vs baseline: 1.0196x; 1.0196x over previous
import jax
import jax.numpy as jnp
from jax.experimental import pallas as pl
from jax.experimental.pallas import tpu as pltpu
from jax.experimental.pallas import tpu_sc as plsc

N = 10000
D = 128
GAMMA = 3.0
T = 1000
K = 5

BM = 400

P_PAIRS = T + K * T + K * T + T + T
GWIN = 128
P_PAD = 13056

CHUNKS = [(0, 2048), (2048, 2048), (4096, 2048), (6144, 2048), (8192, 1808)]


def _mm_body(s_ref, w_ref, o_ref, hidden_ref):
    p = pl.program_id(0)
    mi = pl.program_id(1)

    @pl.when(p == 0)
    def _():
        h = jnp.zeros((BM, D), jnp.float32)
        for c0, cw in CHUNKS:
            s_c = s_ref[:, c0:c0 + cw].astype(jnp.bfloat16)
            w_c = w_ref[c0:c0 + cw, :].astype(jnp.bfloat16)
            h = h + jnp.dot(s_c, w_c, preferred_element_type=jnp.float32)
        hidden_ref[pl.ds(mi * BM, BM), :] = jnp.maximum(
            h, 0.0).astype(jnp.bfloat16)

    @pl.when(p == 1)
    def _():
        o = jnp.zeros((BM, D), jnp.float32)
        for c0, cw in CHUNKS:
            s_c = s_ref[:, c0:c0 + cw].astype(jnp.bfloat16)
            h_c = hidden_ref[c0:c0 + cw, :]
            o = o + jnp.dot(s_c, h_c, preferred_element_type=jnp.float32)
        o_ref[...] = o


def _gcn_out(support, W0):
    return pl.pallas_call(
        _mm_body,
        grid=(2, N // BM),
        in_specs=[
            pl.BlockSpec((BM, N), lambda p, mi: (mi, 0)),
            pl.BlockSpec((N, D), lambda p, mi: (0, 0)),
        ],
        out_specs=pl.BlockSpec((BM, D), lambda p, mi: (mi, 0)),
        out_shape=jax.ShapeDtypeStruct((N, D), jnp.float32),
        scratch_shapes=[
            pltpu.VMEM((N, D), jnp.bfloat16),
        ],
    )(support, W0)


def _sc_gather(out_hbm, left, right):
    left2 = left.reshape(1, P_PAD)
    right2 = right.reshape(1, P_PAD)
    mesh = plsc.VectorSubcoreMesh(core_axis_name="core",
                                  subcore_axis_name="subcore")
    row_t = jax.ShapeDtypeStruct((P_PAD, D), jnp.float32)

    @pl.kernel(out_type=[row_t, row_t], mesh=mesh)
    def k(x_hbm, li_hbm, ri_hbm, lo_hbm, ro_hbm):
        def body(li_vmem, ri_vmem, lo_vmem, ro_vmem):
            pltpu.sync_copy(x_hbm.at[li_vmem.at[0]], lo_vmem)
            pltpu.sync_copy(x_hbm.at[ri_vmem.at[0]], ro_vmem)

        pltpu.emit_pipeline(
            body,
            grid=(P_PAD // GWIN,),
            in_specs=[pl.BlockSpec((1, GWIN), lambda i: (0, i)),
                      pl.BlockSpec((1, GWIN), lambda i: (0, i))],
            out_specs=[pl.BlockSpec((GWIN, D), lambda i: (i, 0)),
                       pl.BlockSpec((GWIN, D), lambda i: (i, 0))],
            core_axis_name=("core", "subcore"),
            dimension_semantics=(pltpu.PARALLEL,),
        )(li_hbm, ri_hbm, lo_hbm, ro_hbm)

    return k(out_hbm, left2, right2)


def _combine_body(l_ref, r_ref, o_ref):
    d = jnp.sum(jnp.abs(l_ref[...] - r_ref[...]), axis=1, keepdims=True)
    dA = d[0:T]
    dA2 = d[11 * T:12 * T]
    dB3 = d[12 * T:13 * T]
    acc = jnp.sum(jnp.maximum(dA2 + GAMMA - dB3, 0.0))
    for k in range(2 * K):
        dBk = d[(1 + k) * T:(2 + k) * T]
        acc = acc + jnp.sum(jnp.maximum(dA + GAMMA - dBk, 0.0))
    o_ref[0, 0] = acc / (2 * K * T + T)


def _combine(L, R):
    return pl.pallas_call(
        _combine_body,
        out_shape=jax.ShapeDtypeStruct((1, 1), jnp.float32),
        out_specs=pl.BlockSpec(memory_space=pltpu.SMEM),
    )(L, R)


def kernel(features, support, W0, ILL0, ILL1, neg_left, neg_right,
           neg2_left, neg2_right, feedback_neg_left, feedback_neg_right,
           feedback_pos_left, feedback_pos_right):
    out = _gcn_out(support, W0)
    pad = jnp.zeros((P_PAD - P_PAIRS,), jnp.int32)
    left = jnp.concatenate(
        [ILL0]
        + [neg_left[k::K] for k in range(K)]
        + [neg2_left[k::K] for k in range(K)]
        + [feedback_pos_left, feedback_neg_left, pad]).astype(jnp.int32)
    right = jnp.concatenate(
        [ILL1]
        + [neg_right[k::K] for k in range(K)]
        + [neg2_right[k::K] for k in range(K)]
        + [feedback_pos_right, feedback_neg_right, pad]).astype(jnp.int32)
    L, R = _sc_gather(out, left, right)
    return _combine(L, R)[0, 0]


# speedup vs baseline: 1.0493x; 1.0291x over previous
import jax
import jax.numpy as jnp
from jax.experimental import pallas as pl
from jax.experimental.pallas import tpu as pltpu
from jax.experimental.pallas import tpu_sc as plsc

N = 10000
D = 128
GAMMA = 3.0
T = 1000
K = 5

BM = 512
NBM = 20

P_PAIRS = T + K * T + K * T + T + T
GWIN = 128
P_PAD = 13056

CHUNKS = [(0, 2048), (2048, 2048), (4096, 2048), (6144, 2048), (8192, 1808)]


def _mm_body(s_ref, w_ref, o_ref, hidden_ref):
    p = pl.program_id(0)
    mi = pl.program_id(1)

    @pl.when(p == 0)
    def _():
        h = jnp.zeros((BM, D), jnp.float32)
        for c0, cw in CHUNKS:
            s_c = s_ref[:, c0:c0 + cw].astype(jnp.bfloat16)
            w_c = w_ref[c0:c0 + cw, :].astype(jnp.bfloat16)
            h = h + jnp.dot(s_c, w_c, preferred_element_type=jnp.float32)
        hidden_ref[pl.ds(mi * BM, BM), :] = jnp.maximum(
            h, 0.0).astype(jnp.bfloat16)

    @pl.when(p == 1)
    def _():
        o = jnp.zeros((BM, D), jnp.float32)
        for c0, cw in CHUNKS:
            s_c = s_ref[:, c0:c0 + cw].astype(jnp.bfloat16)
            h_c = hidden_ref[c0:c0 + cw, :]
            o = o + jnp.dot(s_c, h_c, preferred_element_type=jnp.float32)
        o_ref[...] = o


def _gcn_out(support, W0):
    return pl.pallas_call(
        _mm_body,
        grid=(2, NBM),
        in_specs=[
            pl.BlockSpec((BM, N), lambda p, mi: (mi, 0)),
            pl.BlockSpec((N, D), lambda p, mi: (0, 0)),
        ],
        out_specs=pl.BlockSpec(
            (BM, D), lambda p, mi: (jnp.where(p == 0, 0, mi), 0)),
        out_shape=jax.ShapeDtypeStruct((N, D), jnp.float32),
        scratch_shapes=[
            pltpu.VMEM((NBM * BM, D), jnp.bfloat16),
        ],
    )(support, W0)


def _sc_gather(out_hbm, left, right):
    pad = jnp.zeros((P_PAD - P_PAIRS,), jnp.int32)
    left2 = jnp.concatenate([left, pad]).reshape(1, P_PAD)
    right2 = jnp.concatenate([right, pad]).reshape(1, P_PAD)
    mesh = plsc.VectorSubcoreMesh(core_axis_name="core",
                                  subcore_axis_name="subcore")
    row_t = jax.ShapeDtypeStruct((P_PAD, D), jnp.float32)

    @pl.kernel(out_type=[row_t, row_t], mesh=mesh)
    def k(x_hbm, li_hbm, ri_hbm, lo_hbm, ro_hbm):
        def body(li_vmem, ri_vmem, lo_vmem, ro_vmem):
            pltpu.sync_copy(x_hbm.at[li_vmem.at[0]], lo_vmem)
            pltpu.sync_copy(x_hbm.at[ri_vmem.at[0]], ro_vmem)

        pltpu.emit_pipeline(
            body,
            grid=(P_PAD // GWIN,),
            in_specs=[pl.BlockSpec((1, GWIN), lambda i: (0, i)),
                      pl.BlockSpec((1, GWIN), lambda i: (0, i))],
            out_specs=[pl.BlockSpec((GWIN, D), lambda i: (i, 0)),
                       pl.BlockSpec((GWIN, D), lambda i: (i, 0))],
            core_axis_name=("core", "subcore"),
            dimension_semantics=(pltpu.PARALLEL,),
        )(li_hbm, ri_hbm, lo_hbm, ro_hbm)

    return k(out_hbm, left2, right2)


def _combine_body(l_ref, r_ref, o_ref):
    d = jnp.sum(jnp.abs(l_ref[...] - r_ref[...]), axis=1, keepdims=True)
    dA = d[0:T]
    dA2 = d[11 * T:12 * T]
    dB3 = d[12 * T:13 * T]
    acc = jnp.sum(jnp.maximum(dA2 + GAMMA - dB3, 0.0))
    for k in range(2 * K):
        dBk = d[(1 + k) * T:(2 + k) * T]
        acc = acc + jnp.sum(jnp.maximum(dA + GAMMA - dBk, 0.0))
    o_ref[0, 0] = acc / (2 * K * T + T)


def _combine(L, R):
    return pl.pallas_call(
        _combine_body,
        out_shape=jax.ShapeDtypeStruct((1, 1), jnp.float32),
        out_specs=pl.BlockSpec(memory_space=pltpu.SMEM),
    )(L, R)


def kernel(features, support, W0, ILL0, ILL1, neg_left, neg_right,
           neg2_left, neg2_right, feedback_neg_left, feedback_neg_right,
           feedback_pos_left, feedback_pos_right):
    out = _gcn_out(support, W0)
    nl = neg_left.reshape(T, K).T.reshape(K * T)
    nr = neg_right.reshape(T, K).T.reshape(K * T)
    n2l = neg2_left.reshape(T, K).T.reshape(K * T)
    n2r = neg2_right.reshape(T, K).T.reshape(K * T)
    left = jnp.concatenate([
        ILL0, nl, n2l, feedback_pos_left, feedback_neg_left]).astype(jnp.int32)
    right = jnp.concatenate([
        ILL1, nr, n2r, feedback_pos_right, feedback_neg_right]).astype(jnp.int32)
    L, R = _sc_gather(out, left, right)
    return _combine(L, R)[0, 0]


# speedup vs baseline: 1.0624x; 1.0126x over previous
import jax
import jax.numpy as jnp
from jax.experimental import pallas as pl
from jax.experimental.pallas import tpu as pltpu
from jax.experimental.pallas import tpu_sc as plsc

N = 10000
D = 128
GAMMA = 3.0
T = 1000
K = 5

BM = 400
NBM = 25

P_PAIRS = T + K * T + K * T + T + T
GWIN = 128
P_PAD = 13056

CHUNKS = [(0, 2048), (2048, 2048), (4096, 2048), (6144, 2048), (8192, 1808)]


def _mm_body(s_ref, w_ref, o_ref, hidden_ref):
    p = pl.program_id(0)
    mi = pl.program_id(1)

    @pl.when(p == 0)
    def _():
        h = jnp.zeros((BM, D), jnp.float32)
        for c0, cw in CHUNKS:
            s_c = s_ref[:, c0:c0 + cw].astype(jnp.bfloat16)
            w_c = w_ref[c0:c0 + cw, :].astype(jnp.bfloat16)
            h = h + jnp.dot(s_c, w_c, preferred_element_type=jnp.float32)
        hidden_ref[pl.ds(mi * BM, BM), :] = jnp.maximum(
            h, 0.0).astype(jnp.bfloat16)

    @pl.when(p == 1)
    def _():
        o = jnp.zeros((BM, D), jnp.float32)
        for c0, cw in CHUNKS:
            s_c = s_ref[:, c0:c0 + cw].astype(jnp.bfloat16)
            h_c = hidden_ref[c0:c0 + cw, :]
            o = o + jnp.dot(s_c, h_c, preferred_element_type=jnp.float32)
        o_ref[...] = o


def _gcn_out(support, W0):
    return pl.pallas_call(
        _mm_body,
        grid=(2, NBM),
        in_specs=[
            pl.BlockSpec((BM, N), lambda p, mi: (mi, 0)),
            pl.BlockSpec((N, D), lambda p, mi: (0, 0)),
        ],
        out_specs=pl.BlockSpec(
            (BM, D), lambda p, mi: (jnp.where(p == 0, 0, mi), 0)),
        out_shape=jax.ShapeDtypeStruct((N, D), jnp.float32),
        scratch_shapes=[
            pltpu.VMEM((N, D), jnp.bfloat16),
        ],
    )(support, W0)


def _sc_gather(out_hbm, left, right):
    pad = jnp.zeros((P_PAD - P_PAIRS,), jnp.int32)
    left2 = jnp.concatenate([left, pad]).reshape(1, P_PAD)
    right2 = jnp.concatenate([right, pad]).reshape(1, P_PAD)
    mesh = plsc.VectorSubcoreMesh(core_axis_name="core",
                                  subcore_axis_name="subcore")
    row_t = jax.ShapeDtypeStruct((P_PAD, D), jnp.float32)

    @pl.kernel(out_type=[row_t, row_t], mesh=mesh)
    def k(x_hbm, li_hbm, ri_hbm, lo_hbm, ro_hbm):
        def body(li_vmem, ri_vmem, lo_vmem, ro_vmem):
            pltpu.sync_copy(x_hbm.at[li_vmem.at[0]], lo_vmem)
            pltpu.sync_copy(x_hbm.at[ri_vmem.at[0]], ro_vmem)

        pltpu.emit_pipeline(
            body,
            grid=(P_PAD // GWIN,),
            in_specs=[pl.BlockSpec((1, GWIN), lambda i: (0, i)),
                      pl.BlockSpec((1, GWIN), lambda i: (0, i))],
            out_specs=[pl.BlockSpec((GWIN, D), lambda i: (i, 0)),
                       pl.BlockSpec((GWIN, D), lambda i: (i, 0))],
            core_axis_name=("core", "subcore"),
            dimension_semantics=(pltpu.PARALLEL,),
        )(li_hbm, ri_hbm, lo_hbm, ro_hbm)

    return k(out_hbm, left2, right2)


def _combine_body(l_ref, r_ref, o_ref):
    d = jnp.sum(jnp.abs(l_ref[...] - r_ref[...]), axis=1, keepdims=True)
    dA = d[0:T]
    dA2 = d[11 * T:12 * T]
    dB3 = d[12 * T:13 * T]
    acc = jnp.sum(jnp.maximum(dA2 + GAMMA - dB3, 0.0))
    for k in range(2 * K):
        dBk = d[(1 + k) * T:(2 + k) * T]
        acc = acc + jnp.sum(jnp.maximum(dA + GAMMA - dBk, 0.0))
    o_ref[0, 0] = acc / (2 * K * T + T)


def _combine(L, R):
    return pl.pallas_call(
        _combine_body,
        out_shape=jax.ShapeDtypeStruct((1, 1), jnp.float32),
        out_specs=pl.BlockSpec(memory_space=pltpu.SMEM),
    )(L, R)


def kernel(features, support, W0, ILL0, ILL1, neg_left, neg_right,
           neg2_left, neg2_right, feedback_neg_left, feedback_neg_right,
           feedback_pos_left, feedback_pos_right):
    out = _gcn_out(support, W0)
    nl = neg_left.reshape(T, K).T.reshape(K * T)
    nr = neg_right.reshape(T, K).T.reshape(K * T)
    n2l = neg2_left.reshape(T, K).T.reshape(K * T)
    n2r = neg2_right.reshape(T, K).T.reshape(K * T)
    left = jnp.concatenate([
        ILL0, nl, n2l, feedback_pos_left, feedback_neg_left]).astype(jnp.int32)
    right = jnp.concatenate([
        ILL1, nr, n2r, feedback_pos_right, feedback_neg_right]).astype(jnp.int32)
    L, R = _sc_gather(out, left, right)
    return _combine(L, R)[0, 0]


# speedup vs baseline: 1.0664x; 1.0038x over previous
import jax
import jax.numpy as jnp
from jax.experimental import pallas as pl
from jax.experimental.pallas import tpu as pltpu
from jax.experimental.pallas import tpu_sc as plsc

N = 10000
D = 128
GAMMA = 3.0
T = 1000
K = 5

BM = 400
NBM = 25

P_PAIRS = T + K * T + K * T + T + T
GWIN = 128
P_PAD = 13056

CHUNKS = [(0, 2048), (2048, 2048), (4096, 2048), (6144, 2048), (8192, 1808)]


def _mm_body(s_ref, w_ref, o_ref, hidden_ref):
    p = pl.program_id(0)
    mi = pl.program_id(1)

    @pl.when(p == 0)
    def _():
        h = jnp.zeros((BM, D), jnp.float32)
        for c0, cw in CHUNKS:
            s_c = s_ref[:, c0:c0 + cw].astype(jnp.bfloat16)
            w_c = w_ref[c0:c0 + cw, :].astype(jnp.bfloat16)
            h = h + jnp.dot(s_c, w_c, preferred_element_type=jnp.float32)
        hidden_ref[pl.ds(mi * BM, BM), :] = jnp.maximum(
            h, 0.0).astype(jnp.bfloat16)

    @pl.when(p == 1)
    def _():
        o = jnp.zeros((BM, D), jnp.float32)
        for c0, cw in CHUNKS:
            s_c = s_ref[:, c0:c0 + cw].astype(jnp.bfloat16)
            h_c = hidden_ref[c0:c0 + cw, :]
            o = o + jnp.dot(s_c, h_c, preferred_element_type=jnp.float32)
        o_ref[...] = o


def _gcn_out(support, W0):
    return pl.pallas_call(
        _mm_body,
        grid=(2, NBM),
        in_specs=[
            pl.BlockSpec((BM, N), lambda p, mi: (mi, 0)),
            pl.BlockSpec((N, D), lambda p, mi: (0, 0)),
        ],
        out_specs=pl.BlockSpec(
            (BM, D), lambda p, mi: (jnp.where(p == 0, 0, mi), 0)),
        out_shape=jax.ShapeDtypeStruct((N, D), jnp.float32),
        scratch_shapes=[
            pltpu.VMEM((N, D), jnp.bfloat16),
        ],
    )(support, W0)


def _sc_gather(out_hbm, left, right):
    pad = jnp.zeros((P_PAD - P_PAIRS,), jnp.int32)
    left2 = jnp.concatenate([left, pad]).reshape(1, P_PAD)
    right2 = jnp.concatenate([right, pad]).reshape(1, P_PAD)
    mesh = plsc.VectorSubcoreMesh(core_axis_name="core",
                                  subcore_axis_name="subcore")
    row_t = jax.ShapeDtypeStruct((P_PAD, D), jnp.float32)

    @pl.kernel(out_type=[row_t, row_t], mesh=mesh)
    def k(x_hbm, li_hbm, ri_hbm, lo_hbm, ro_hbm):
        def body(li_vmem, ri_vmem, lo_vmem, ro_vmem):
            pltpu.sync_copy(x_hbm.at[li_vmem.at[0]], lo_vmem)
            pltpu.sync_copy(x_hbm.at[ri_vmem.at[0]], ro_vmem)

        pltpu.emit_pipeline(
            body,
            grid=(P_PAD // GWIN,),
            in_specs=[pl.BlockSpec((1, GWIN), lambda i: (0, i)),
                      pl.BlockSpec((1, GWIN), lambda i: (0, i))],
            out_specs=[pl.BlockSpec((GWIN, D), lambda i: (i, 0)),
                       pl.BlockSpec((GWIN, D), lambda i: (i, 0))],
            core_axis_name=("core", "subcore"),
            dimension_semantics=(pltpu.PARALLEL,),
        )(li_hbm, ri_hbm, lo_hbm, ro_hbm)

    return k(out_hbm, left2, right2)


def _combine_body(l_ref, r_ref, o_ref):
    d = jnp.sum(jnp.abs(l_ref[...] - r_ref[...]), axis=1, keepdims=True)
    dA = d[0:T]
    dA2 = d[11 * T:12 * T]
    dB3 = d[12 * T:13 * T]
    acc = jnp.sum(jnp.maximum(dA2 + GAMMA - dB3, 0.0))
    for k in range(2 * K):
        dBk = d[(1 + k) * T:(2 + k) * T]
        acc = acc + jnp.sum(jnp.maximum(dA + GAMMA - dBk, 0.0))
    o_ref[0, 0] = acc / (2 * K * T + T)


def _combine(L, R):
    return pl.pallas_call(
        _combine_body,
        out_shape=jax.ShapeDtypeStruct((1, 1), jnp.float32),
        out_specs=pl.BlockSpec(memory_space=pltpu.SMEM),
    )(L, R)


def kernel(features, support, W0, ILL0, ILL1, neg_left, neg_right,
           neg2_left, neg2_right, feedback_neg_left, feedback_neg_right,
           feedback_pos_left, feedback_pos_right):
    out = _gcn_out(support, W0)
    negs = jnp.stack([neg_left, neg_right, neg2_left, neg2_right])
    negs_km = jnp.transpose(negs.reshape(4, T, K), (0, 2, 1)).reshape(4, K * T)
    nl, nr, n2l, n2r = negs_km[0], negs_km[1], negs_km[2], negs_km[3]
    left = jnp.concatenate([
        ILL0, nl, n2l, feedback_pos_left, feedback_neg_left]).astype(jnp.int32)
    right = jnp.concatenate([
        ILL1, nr, n2r, feedback_pos_right, feedback_neg_right]).astype(jnp.int32)
    L, R = _sc_gather(out, left, right)
    return _combine(L, R)[0, 0]


# speedup vs baseline: 1.0673x; 1.0008x over previous
import jax
import jax.numpy as jnp
from jax.experimental import pallas as pl
from jax.experimental.pallas import tpu as pltpu
from jax.experimental.pallas import tpu_sc as plsc

N = 10000
D = 128
GAMMA = 3.0
T = 1000
K = 5

BM = 400
NBM = 25

P_PAIRS = T + K * T + K * T + T + T
GWIN = 128
P_PAD = 13056

CHUNKS = [(0, 2048), (2048, 2048), (4096, 2048), (6144, 2048), (8192, 1808)]


def _mm_body(s_ref, w_ref, o_ref, hidden_ref):
    p = pl.program_id(0)
    mi = pl.program_id(1)

    @pl.when(p == 0)
    def _():
        h = jnp.zeros((BM, D), jnp.float32)
        for c0, cw in CHUNKS:
            s_c = s_ref[:, c0:c0 + cw].astype(jnp.bfloat16)
            w_c = w_ref[c0:c0 + cw, :].astype(jnp.bfloat16)
            h = h + jnp.dot(s_c, w_c, preferred_element_type=jnp.float32)
        hidden_ref[pl.ds(mi * BM, BM), :] = jnp.maximum(
            h, 0.0).astype(jnp.bfloat16)

    @pl.when(p == 1)
    def _():
        o = jnp.zeros((BM, D), jnp.float32)
        for c0, cw in CHUNKS:
            s_c = s_ref[:, c0:c0 + cw].astype(jnp.bfloat16)
            h_c = hidden_ref[c0:c0 + cw, :]
            o = o + jnp.dot(s_c, h_c, preferred_element_type=jnp.float32)
        o_ref[...] = o


def _gcn_out(support, W0):
    return pl.pallas_call(
        _mm_body,
        grid=(2, NBM),
        in_specs=[
            pl.BlockSpec((BM, N), lambda p, mi: (mi, 0)),
            pl.BlockSpec((N, D), lambda p, mi: (0, 0)),
        ],
        out_specs=pl.BlockSpec(
            (BM, D), lambda p, mi: (jnp.where(p == 0, 0, mi), 0)),
        out_shape=jax.ShapeDtypeStruct((N, D), jnp.float32),
        scratch_shapes=[
            pltpu.VMEM((N, D), jnp.bfloat16),
        ],
    )(support, W0)


def _sc_gather(out_hbm, left, right):
    pad = jnp.zeros((P_PAD - P_PAIRS,), jnp.int32)
    left2 = jnp.concatenate([left, pad]).reshape(1, P_PAD)
    right2 = jnp.concatenate([right, pad]).reshape(1, P_PAD)
    mesh = plsc.VectorSubcoreMesh(core_axis_name="core",
                                  subcore_axis_name="subcore")
    row_t = jax.ShapeDtypeStruct((P_PAD, D), jnp.float32)

    @pl.kernel(out_type=[row_t, row_t], mesh=mesh)
    def k(x_hbm, li_hbm, ri_hbm, lo_hbm, ro_hbm):
        def body(li_vmem, ri_vmem, lo_vmem, ro_vmem):
            pltpu.sync_copy(x_hbm.at[li_vmem.at[0]], lo_vmem)
            pltpu.sync_copy(x_hbm.at[ri_vmem.at[0]], ro_vmem)

        pltpu.emit_pipeline(
            body,
            grid=(P_PAD // GWIN,),
            in_specs=[pl.BlockSpec((1, GWIN), lambda i: (0, i)),
                      pl.BlockSpec((1, GWIN), lambda i: (0, i))],
            out_specs=[pl.BlockSpec((GWIN, D), lambda i: (i, 0)),
                       pl.BlockSpec((GWIN, D), lambda i: (i, 0))],
            core_axis_name=("core", "subcore"),
            dimension_semantics=(pltpu.PARALLEL,),
        )(li_hbm, ri_hbm, lo_hbm, ro_hbm)

    return k(out_hbm, left2, right2)


def _combine_body(l_ref, r_ref, o_ref):
    d = jnp.sum(jnp.abs(l_ref[...] - r_ref[...]), axis=1, keepdims=True)
    dA = d[0:T]
    dA2 = d[11 * T:12 * T]
    dB3 = d[12 * T:13 * T]
    acc = jnp.sum(jnp.maximum(dA2 + GAMMA - dB3, 0.0))
    for k in range(2 * K):
        dBk = d[(1 + k) * T:(2 + k) * T]
        acc = acc + jnp.sum(jnp.maximum(dA + GAMMA - dBk, 0.0))
    o_ref[...] = acc / (2 * K * T + T)


def _combine(L, R):
    return pl.pallas_call(
        _combine_body,
        out_shape=jax.ShapeDtypeStruct((), jnp.float32),
        out_specs=pl.BlockSpec(memory_space=pltpu.SMEM),
    )(L, R)


def kernel(features, support, W0, ILL0, ILL1, neg_left, neg_right,
           neg2_left, neg2_right, feedback_neg_left, feedback_neg_right,
           feedback_pos_left, feedback_pos_right):
    out = _gcn_out(support, W0)
    negs = jnp.stack([neg_left, neg_right, neg2_left, neg2_right])
    negs_km = jnp.transpose(negs.reshape(4, T, K), (0, 2, 1)).reshape(4, K * T)
    nl, nr, n2l, n2r = negs_km[0], negs_km[1], negs_km[2], negs_km[3]
    left = jnp.concatenate([
        ILL0, nl, n2l, feedback_pos_left, feedback_neg_left]).astype(jnp.int32)
    right = jnp.concatenate([
        ILL1, nr, n2r, feedback_pos_right, feedback_neg_right]).astype(jnp.int32)
    L, R = _sc_gather(out, left, right)
    return _combine(L, R)
